# Initial kernel scaffold; baseline (speedup 1.0000x reference)
import jax
import jax.numpy as jnp
from jax.experimental import pallas as pl


def kernel(predictions, targets, imgs):
    raise NotImplementedError("write your pallas kernel here")


# speedup vs baseline: 2.8318x; 2.8318x over previous
import functools

import jax
import jax.numpy as jnp
from jax.experimental import pallas as pl

_NUM_CLASSES = 80
_LC = 5.0
_LN = 0.5
_ANCHORS = (
    (1.3221, 1.73145),
    (3.19275, 4.00944),
    (5.05587, 8.09892),
    (9.47112, 4.84053),
    (11.2364, 10.0071),
)
_A = 5
_S = 13
_G = 30


def _sigmoid(x):
    return 1.0 / (1.0 + jnp.exp(-x))


def _loss_kernel(pred_ref, tgt_ref, tot_ref, coord_ref, obj_ref, noobj_ref):
    b = pl.program_id(0)

    zero = jnp.zeros((1, 1), jnp.float32)

    @pl.when(b == 0)
    def _init():
        tot_ref[...] = zero
        coord_ref[...] = zero
        obj_ref[...] = zero
        noobj_ref[...] = zero

    p = pred_ref[0]

    s_all = 0.0
    for a in range(_A):
        conf = _sigmoid(p[:, 85 * a + 4 : 85 * a + 5])
        s_all = s_all + jnp.sum(conf * conf)

    t = tgt_ref[0]
    gx = t[:, 0:1]
    gy = t[:, 1:2]
    gw = t[:, 2:3]
    gh = t[:, 3:4]
    gt_w = gw * _S
    gt_h = gh * _S

    best_iou = jnp.full((_G, 1), -1.0, jnp.float32)
    best_a = jnp.zeros((_G, 1), jnp.int32)
    for a, (aw, ah) in enumerate(_ANCHORS):
        inter = jnp.minimum(gt_w, aw) * jnp.minimum(gt_h, ah)
        union = gt_w * gt_h + aw * ah - inter
        iou_a = jnp.where(union > 0, inter / jnp.where(union > 0, union, 1.0), 0.0)
        upd = iou_a > best_iou
        best_iou = jnp.where(upd, iou_a, best_iou)
        best_a = jnp.where(upd, a, best_a)

    gi = jnp.clip((gx * _S).astype(jnp.int32), 0, _S - 1)
    gj = jnp.clip((gy * _S).astype(jnp.int32), 0, _S - 1)
    row = gj * _S + gi

    iota_row = jax.lax.broadcasted_iota(jnp.int32, (_G, _S * _S), 1)
    rowoh = (iota_row == row).astype(jnp.float32)
    gat = jnp.dot(rowoh, p, preferred_element_type=jnp.float32)

    tx = jnp.zeros((_G, 1), jnp.float32)
    ty = jnp.zeros((_G, 1), jnp.float32)
    tw = jnp.zeros((_G, 1), jnp.float32)
    th = jnp.zeros((_G, 1), jnp.float32)
    to = jnp.zeros((_G, 1), jnp.float32)
    aw_g = jnp.zeros((_G, 1), jnp.float32)
    ah_g = jnp.zeros((_G, 1), jnp.float32)
    for a, (aw, ah) in enumerate(_ANCHORS):
        sel = (best_a == a).astype(jnp.float32)
        tx = tx + sel * gat[:, 85 * a + 0 : 85 * a + 1]
        ty = ty + sel * gat[:, 85 * a + 1 : 85 * a + 2]
        tw = tw + sel * gat[:, 85 * a + 2 : 85 * a + 3]
        th = th + sel * gat[:, 85 * a + 3 : 85 * a + 4]
        to = to + sel * gat[:, 85 * a + 4 : 85 * a + 5]
        aw_g = aw_g + sel * aw
        ah_g = ah_g + sel * ah

    pbx = _sigmoid(tx)
    pby = _sigmoid(ty)
    pbw = jnp.exp(tw) * aw_g
    pbh = jnp.exp(th) * ah_g
    conf_c = _sigmoid(to)

    gi_f = gi.astype(jnp.float32)
    gj_f = gj.astype(jnp.float32)
    ggx = gx * _S - gi_f
    ggy = gy * _S - gj_f
    ggw = gt_w
    ggh = gt_h

    cx_gt = gx * _S
    cy_gt = gy * _S
    cx_pr = pbx + gi_f
    cy_pr = pby + gj_f
    iw = jnp.maximum(
        0.0, jnp.minimum(cx_gt + ggw / 2, cx_pr + pbw / 2) - jnp.maximum(cx_gt - ggw / 2, cx_pr - pbw / 2)
    )
    ih = jnp.maximum(
        0.0, jnp.minimum(cy_gt + ggh / 2, cy_pr + pbh / 2) - jnp.maximum(cy_gt - ggh / 2, cy_pr - pbh / 2)
    )
    inter_a = iw * ih
    union_a = ggw * ggh + pbw * pbh - inter_a
    iou = jnp.where(union_a > 0, inter_a / jnp.where(union_a > 0, union_a, 1.0), 0.0)

    cell = best_a * (_S * _S) + row
    cell_row = cell.reshape(1, _G)
    same = cell == cell_row
    later = jax.lax.broadcasted_iota(jnp.int32, (_G, _G), 1) > jax.lax.broadcasted_iota(
        jnp.int32, (_G, _G), 0
    )
    dup = jnp.sum(jnp.where(same & later, 1.0, 0.0), axis=1, keepdims=True)
    kept = (dup == 0.0).astype(jnp.float32)

    coord_t = (pbx - ggx) ** 2 + (pby - ggy) ** 2
    coord_t = coord_t + (jnp.sqrt(pbw + 1e-6) - jnp.sqrt(ggw + 1e-6)) ** 2
    coord_t = coord_t + (jnp.sqrt(pbh + 1e-6) - jnp.sqrt(ggh + 1e-6)) ** 2
    obj_t = (iou - conf_c) ** 2
    noobj_t = conf_c * conf_c

    coord_ref[...] += (_LC * jnp.sum(kept * coord_t)).reshape(1, 1)
    obj_ref[...] += jnp.sum(kept * obj_t).reshape(1, 1)
    noobj_ref[...] += (s_all - jnp.sum(kept * noobj_t)).reshape(1, 1)

    @pl.when(b == pl.num_programs(0) - 1)
    def _fin():
        lnoobj = _LN * noobj_ref[...]
        tot_ref[...] = _LC * coord_ref[...] + obj_ref[...] + _LN * lnoobj
        noobj_ref[...] = lnoobj


def kernel(predictions, targets, imgs):
    del imgs
    B = predictions.shape[0]
    p3 = predictions.reshape(B, _S * _S, 425)
    t3 = targets
    scal = jax.ShapeDtypeStruct((1, 1), jnp.float32)
    out = pl.pallas_call(
        _loss_kernel,
        grid=(B,),
        in_specs=[
            pl.BlockSpec((1, _S * _S, 425), lambda b: (b, 0, 0)),
            pl.BlockSpec((1, _G, 5), lambda b: (b, 0, 0)),
        ],
        out_specs=[pl.BlockSpec((1, 1), lambda b: (0, 0))] * 4,
        out_shape=[scal, scal, scal, scal],
    )(p3, t3)
    total, coord, lobj, lnoobj = (o[0, 0] for o in out)
    return (total, coord, lobj + lnoobj)
